# baseline (speedup 1.0000x reference)
import functools

import jax
import jax.numpy as jnp
from jax import lax
from jax.experimental import pallas as pl
from jax.experimental.pallas import tpu as pltpu
from jax.experimental.pallas import tpu_sc as plsc

_N = 65536

_NC = 2
_NS = 16
_NW = _NC * _NS
_L = 16
_COLS = _N // _NW
_CHUNK = 3 * _COLS

_mesh = plsc.VectorSubcoreMesh(core_axis_name="c", subcore_axis_name="s")


@functools.partial(
    pl.kernel,
    mesh=_mesh,
    out_type=jax.ShapeDtypeStruct((3, _N), jnp.float32),
    scratch_types=[
        pltpu.VMEM((3, _COLS), jnp.float32),
        pltpu.VMEM((3, _COLS), jnp.int32),
    ],
)
def _add_sc(x_hbm, i_hbm, o_hbm, xv, iv):
    wid = lax.axis_index("s") * _NC + lax.axis_index("c")
    base = wid * _COLS
    pltpu.sync_copy(x_hbm.at[:, pl.ds(base, _COLS)], xv)
    pltpu.sync_copy(i_hbm.at[:, pl.ds(base, _COLS)], iv)

    def step(j, carry):
        s = pl.ds(j * _L, _L)
        for r in range(3):
            xv[r, s] = xv[r, s] + iv[r, s].astype(jnp.float32)
        return carry

    lax.fori_loop(0, _COLS // _L, step, 0)
    pltpu.sync_copy(xv, o_hbm.at[:, pl.ds(base, _COLS)])


def kernel(input_xyzs, query_xyz_index):
    out = _add_sc(input_xyzs.T, query_xyz_index.T)
    return out.T


# speedup vs baseline: 8.8933x; 8.8933x over previous
import jax
import jax.numpy as jnp
from jax.experimental import pallas as pl
from jax.experimental.pallas import tpu as pltpu

_N = 65536
_GRID = 2
_BLK = _N // _GRID


def _add_body(x_ref, i_ref, o_ref):
    o_ref[...] = x_ref[...] + i_ref[...].astype(jnp.float32)


def kernel(input_xyzs, query_xyz_index):
    x = input_xyzs.T
    i = query_xyz_index.T
    out = pl.pallas_call(
        _add_body,
        grid=(_GRID,),
        in_specs=[
            pl.BlockSpec((3, _BLK), lambda g: (0, g)),
            pl.BlockSpec((3, _BLK), lambda g: (0, g)),
        ],
        out_specs=pl.BlockSpec((3, _BLK), lambda g: (0, g)),
        out_shape=jax.ShapeDtypeStruct((3, _N), jnp.float32),
        compiler_params=pltpu.CompilerParams(
            dimension_semantics=("parallel",),
        ),
    )(x, i)
    return out.T
